# Initial kernel scaffold; baseline (speedup 1.0000x reference)
import jax
import jax.numpy as jnp
from jax.experimental import pallas as pl


def kernel(x, edge_index, iso_idx, iso_embed, W_in, b_in, Wl1, bl1, Wr1, g1, be1, Wl2, bl2, Wr2, g2, be2, Wl3, bl3, Wr3, g3, be3, Wl4, bl4, Wr4, g4, be4, Wh1, bh1, gh, beh, Wh2, bh2):
    raise NotImplementedError("write your pallas kernel here")


# speedup vs baseline: 2.4785x; 2.4785x over previous
import functools

import jax
import jax.numpy as jnp
from jax import lax
from jax.experimental import pallas as pl
from jax.experimental.pallas import tpu as pltpu
from jax.experimental.pallas import tpu_sc as plsc

N = 10000
E = 320000
D = 128
H = 256
EMB = 8
ISO = 16
C = 32

CHUNK = 128
NCHUNKS = 2560
EP = NCHUNKS * CHUNK
NSUB = 16
CH_PER_SUB = NCHUNKS // NSUB
NACC = 10112
ZROWS = NACC // NSUB

BN = 2000
GRID = N // BN


def _seg_body(h2, srcr, dstr, zrows, s_out, src_v, dst_v, rows_v, acc_sh, sem):
    c = lax.axis_index("c")
    s = lax.axis_index("s")

    pltpu.sync_copy(zrows, acc_sh.at[pl.ds(s * ZROWS, ZROWS)])
    plsc.subcore_barrier()

    @pl.loop(0, CH_PER_SUB)
    def chunk_body(j):
        g = s * CH_PER_SUB + j
        pltpu.sync_copy(srcr.at[c * NCHUNKS + g], src_v)
        pltpu.sync_copy(dstr.at[g], dst_v)
        pltpu.async_copy(h2.at[src_v], rows_v, sem).wait()
        pltpu.sync_copy(rows_v, acc_sh.at[dst_v], add=True)

    plsc.subcore_barrier()

    pltpu.sync_copy(acc_sh.at[pl.ds(s * ZROWS, ZROWS)],
                    s_out.at[c, pl.ds(s * ZROWS, ZROWS)])


def _make_seg_kernel():
    mesh = plsc.VectorSubcoreMesh(core_axis_name="c", subcore_axis_name="s")
    return pl.kernel(
        _seg_body,
        out_type=jax.ShapeDtypeStruct((2, NACC, D), jnp.float32),
        mesh=mesh,
        scratch_types=[
            pltpu.VMEM((CHUNK,), jnp.int32),
            pltpu.VMEM((CHUNK,), jnp.int32),
            pltpu.VMEM((CHUNK, D), jnp.float32),
            pltpu.VMEM_SHARED((NACC, D), jnp.float32),
            pltpu.SemaphoreType.DMA,
        ],
        name="sc_segsum",
    )


CNT_PER_W = NCHUNKS // 32


def _cnt_body(dstr, zrows, ones_hbm, cnt_out, dst_v, ones_v, acc_sh, sem):
    c = lax.axis_index("c")
    s = lax.axis_index("s")
    w = c * NSUB + s

    pltpu.sync_copy(zrows, acc_sh.at[pl.ds(s * ZROWS, ZROWS)])
    pltpu.sync_copy(ones_hbm, ones_v)
    plsc.subcore_barrier()

    @pl.loop(0, CNT_PER_W)
    def chunk_body(j):
        g = w * CNT_PER_W + j
        pltpu.sync_copy(dstr.at[g], dst_v)
        pltpu.sync_copy(ones_v, acc_sh.at[dst_v], add=True)

    plsc.subcore_barrier()
    pltpu.sync_copy(acc_sh.at[pl.ds(s * ZROWS, ZROWS)],
                    cnt_out.at[c, pl.ds(s * ZROWS, ZROWS)])


def _make_cnt_kernel():
    mesh = plsc.VectorSubcoreMesh(core_axis_name="c", subcore_axis_name="s")
    return pl.kernel(
        _cnt_body,
        out_type=jax.ShapeDtypeStruct((2, NACC, D), jnp.float32),
        mesh=mesh,
        scratch_types=[
            pltpu.VMEM((CHUNK,), jnp.int32),
            pltpu.VMEM((CHUNK, D), jnp.float32),
            pltpu.VMEM_SHARED((NACC, D), jnp.float32),
            pltpu.SemaphoreType.DMA,
        ],
        name="sc_degcount",
    )


_SQRT1_2 = 0.7071067811865476


def _gelu(z):
    return 0.5 * z * (1.0 + lax.erf(z * _SQRT1_2))


def _embed_body(x_ref, iso_ref, isoemb_ref, win_ref, bin_ref, out_ref):
    x = x_ref[...]
    t = isoemb_ref[...] @ win_ref[D:D + EMB, :]
    oh = (iso_ref[...] == lax.broadcasted_iota(jnp.int32, (BN, ISO), 1))
    h0 = x @ win_ref[:D, :] + oh.astype(jnp.float32) @ t + bin_ref[...]
    out_ref[0] = h0[:, :D]
    out_ref[1] = h0[:, D:]


def _layer_body(hp_ref, s_ref, cnt_ref, wl_ref, bl_ref, wr_ref, g_ref, be_ref,
                out_ref):
    hp = jnp.concatenate([hp_ref[0], hp_ref[1]], axis=-1)
    sv = jnp.concatenate([s_ref[0], s_ref[1]], axis=-1)
    cnt = cnt_ref[0][:, 0:1] + cnt_ref[1][:, 0:1]
    agg = sv / jnp.maximum(cnt, 1.0)
    z = agg @ wl_ref[...] + bl_ref[...] + hp @ wr_ref[...]
    m = jnp.mean(z, axis=-1, keepdims=True)
    v = jnp.mean((z - m) ** 2, axis=-1, keepdims=True)
    z = (z - m) * lax.rsqrt(v + 1e-5) * g_ref[...] + be_ref[...]
    res = hp + _gelu(z)
    out_ref[0] = res[:, :D]
    out_ref[1] = res[:, D:]


def _head_body(hp_ref, wh1_ref, bh1_ref, gh_ref, beh_ref, wh2_ref, bh2_ref,
               out_ref):
    hp = jnp.concatenate([hp_ref[0], hp_ref[1]], axis=-1)
    z = hp @ wh1_ref[...] + bh1_ref[...]
    m = jnp.mean(z, axis=-1, keepdims=True)
    v = jnp.mean((z - m) ** 2, axis=-1, keepdims=True)
    z = (z - m) * lax.rsqrt(v + 1e-5) * gh_ref[...] + beh_ref[...]
    out_ref[...] = _gelu(z) @ wh2_ref[...] + bh2_ref[...]


def _stacked_spec():
    return pl.BlockSpec((2, BN, D), lambda i: (0, i, 0))


def _full_spec(shape):
    nd = len(shape)
    return pl.BlockSpec(shape, lambda i: (0,) * nd)


_embed_call = pl.pallas_call(
    _embed_body,
    grid=(GRID,),
    in_specs=[
        pl.BlockSpec((BN, D), lambda i: (i, 0)),
        pl.BlockSpec((BN, 1), lambda i: (i, 0)),
        _full_spec((ISO, EMB)),
        _full_spec((D + EMB, H)),
        _full_spec((1, H)),
    ],
    out_specs=_stacked_spec(),
    out_shape=jax.ShapeDtypeStruct((2, N, D), jnp.float32),
)

_layer_call = pl.pallas_call(
    _layer_body,
    grid=(GRID,),
    in_specs=[
        _stacked_spec(),
        _stacked_spec(),
        pl.BlockSpec((2, BN, 8), lambda i: (0, i, 0)),
        _full_spec((H, H)),
        _full_spec((1, H)),
        _full_spec((H, H)),
        _full_spec((1, H)),
        _full_spec((1, H)),
    ],
    out_specs=_stacked_spec(),
    out_shape=jax.ShapeDtypeStruct((2, N, D), jnp.float32),
)

_head_call = pl.pallas_call(
    _head_body,
    grid=(GRID,),
    in_specs=[
        _stacked_spec(),
        _full_spec((H, D)),
        _full_spec((1, D)),
        _full_spec((1, D)),
        _full_spec((1, D)),
        _full_spec((D, C)),
        _full_spec((1, C)),
    ],
    out_specs=pl.BlockSpec((BN, C), lambda i: (i, 0)),
    out_shape=jax.ShapeDtypeStruct((N, C), jnp.float32),
)

_seg = _make_seg_kernel()
_cnt = _make_cnt_kernel()


def kernel(x, edge_index, iso_idx, iso_embed, W_in, b_in,
           Wl1, bl1, Wr1, g1, be1, Wl2, bl2, Wr2, g2, be2,
           Wl3, bl3, Wr3, g3, be3, Wl4, bl4, Wr4, g4, be4,
           Wh1, bh1, gh, beh, Wh2, bh2):
    src = edge_index[0].astype(jnp.int32)
    dst = edge_index[1].astype(jnp.int32)
    pad = EP - E
    src_p = jnp.concatenate([src, jnp.zeros((pad,), jnp.int32)])
    srcr = jnp.concatenate([src_p, src_p + N]).reshape(2 * NCHUNKS, CHUNK)
    dstr = jnp.concatenate([dst, jnp.full((pad,), N, jnp.int32)]).reshape(
        NCHUNKS, CHUNK)
    zrows = jnp.zeros((ZROWS, D), jnp.float32)
    ones128 = jnp.ones((CHUNK, D), jnp.float32)

    iso2d = iso_idx.astype(jnp.int32).reshape(N, 1)
    b_in2 = b_in.reshape(1, H)

    h = _embed_call(x, iso2d, iso_embed, W_in, b_in2)
    cnt_full = _cnt(dstr, zrows, ones128)
    cnt = cnt_full[:, :, :8]

    layer_params = [
        (Wl1, bl1, Wr1, g1, be1),
        (Wl2, bl2, Wr2, g2, be2),
        (Wl3, bl3, Wr3, g3, be3),
        (Wl4, bl4, Wr4, g4, be4),
    ]
    for (Wl, bl, Wr, g, be) in layer_params:
        h2 = h.reshape(2 * N, D)
        s = _seg(h2, srcr, dstr, zrows)
        h = _layer_call(h, s, cnt, Wl, bl.reshape(1, H), Wr,
                        g.reshape(1, H), be.reshape(1, H))

    return _head_call(h, Wh1, bh1.reshape(1, D), gh.reshape(1, D),
                      beh.reshape(1, D), Wh2, bh2.reshape(1, C))


# speedup vs baseline: 3.2701x; 1.3194x over previous
import functools

import jax
import jax.numpy as jnp
from jax import lax
from jax.experimental import pallas as pl
from jax.experimental.pallas import tpu as pltpu
from jax.experimental.pallas import tpu_sc as plsc

N = 10000
E = 320000
D = 128
H = 256
EMB = 8
ISO = 16
C = 32

CHUNK = 128
NCHUNKS = 2560
EP = NCHUNKS * CHUNK
NSUB = 16
CH_PER_SUB = NCHUNKS // NSUB
NACC = 10112
ZROWS = NACC // NSUB

BN = 2000
GRID = N // BN


NBUF = 2
WIN = 40
NWIN = CH_PER_SUB // WIN
WROUNDS = WIN // NBUF


def _seg_body(h2, srcr, dstr, zrows, s_out, src_win, dst_win,
              r0, r1, acc_sh, g0, g1, s0, s1):
    c = lax.axis_index("c")
    s = lax.axis_index("s")
    rows = (r0, r1)
    gsem = (g0, g1)
    ssem = (s0, s1)

    pltpu.sync_copy(zrows, acc_sh.at[pl.ds(s * ZROWS, ZROWS)])
    plsc.subcore_barrier()

    def fire_gather(b, jj):
        pltpu.async_copy(h2.at[src_win.at[jj]], rows[b], gsem[b])

    def wait_gather(b, jj):
        pltpu.make_async_copy(h2.at[src_win.at[jj]], rows[b], gsem[b]).wait()

    def fire_scatter(b, jj):
        pltpu.async_copy(rows[b], acc_sh.at[dst_win.at[jj]], ssem[b],
                         add=True)

    def wait_scatter(b, jj):
        pltpu.make_async_copy(rows[b], acc_sh.at[dst_win.at[jj]],
                              ssem[b]).wait()

    @pl.loop(0, NWIN)
    def window_body(w):
        base = s * CH_PER_SUB + w * WIN
        pltpu.sync_copy(srcr.at[pl.ds(c * NCHUNKS + base, WIN)], src_win)
        pltpu.sync_copy(dstr.at[pl.ds(base, WIN)], dst_win)

        for b in range(NBUF):
            fire_gather(b, b)

        @pl.loop(0, WROUNDS - 1)
        def round_body(r):
            j0 = r * NBUF
            for b in range(NBUF):
                wait_gather(b, j0 + b)
                fire_scatter(b, j0 + b)
            for b in range(NBUF):
                wait_scatter(b, j0 + b)
                fire_gather(b, j0 + NBUF + b)

        j0 = (WROUNDS - 1) * NBUF
        for b in range(NBUF):
            wait_gather(b, j0 + b)
            fire_scatter(b, j0 + b)
        for b in range(NBUF):
            wait_scatter(b, j0 + b)

    plsc.subcore_barrier()

    pltpu.sync_copy(acc_sh.at[pl.ds(s * ZROWS, ZROWS)],
                    s_out.at[c, pl.ds(s * ZROWS, ZROWS)])


def _make_seg_kernel():
    mesh = plsc.VectorSubcoreMesh(core_axis_name="c", subcore_axis_name="s")
    return pl.kernel(
        _seg_body,
        out_type=jax.ShapeDtypeStruct((2, NACC, D), jnp.float32),
        mesh=mesh,
        scratch_types=[
            pltpu.VMEM((WIN, CHUNK), jnp.int32),
            pltpu.VMEM((WIN, CHUNK), jnp.int32),
            pltpu.VMEM((CHUNK, D), jnp.float32),
            pltpu.VMEM((CHUNK, D), jnp.float32),
            pltpu.VMEM_SHARED((NACC, D), jnp.float32),
            pltpu.SemaphoreType.DMA,
            pltpu.SemaphoreType.DMA,
            pltpu.SemaphoreType.DMA,
            pltpu.SemaphoreType.DMA,
        ],
        name="sc_segsum",
    )


CNT_PER_W = NCHUNKS // 32


def _cnt_body(dstr, zrows, ones_hbm, cnt_out, dst_all, ones_v, acc_sh, sem):
    c = lax.axis_index("c")
    s = lax.axis_index("s")
    w = c * NSUB + s

    pltpu.sync_copy(zrows, acc_sh.at[pl.ds(s * ZROWS, ZROWS)])
    pltpu.sync_copy(ones_hbm, ones_v)
    pltpu.sync_copy(dstr.at[pl.ds(w * CNT_PER_W, CNT_PER_W)], dst_all)
    plsc.subcore_barrier()

    @pl.loop(0, CNT_PER_W // 4)
    def chunk_body(r):
        j0 = r * 4
        for b in range(4):
            pltpu.async_copy(ones_v, acc_sh.at[dst_all.at[j0 + b]], sem,
                             add=True)
        for b in range(4):
            pltpu.make_async_copy(ones_v, acc_sh.at[dst_all.at[j0 + b]],
                                  sem).wait()

    plsc.subcore_barrier()
    pltpu.sync_copy(acc_sh.at[pl.ds(s * ZROWS, ZROWS)],
                    cnt_out.at[c, pl.ds(s * ZROWS, ZROWS)])


def _make_cnt_kernel():
    mesh = plsc.VectorSubcoreMesh(core_axis_name="c", subcore_axis_name="s")
    return pl.kernel(
        _cnt_body,
        out_type=jax.ShapeDtypeStruct((2, NACC, D), jnp.float32),
        mesh=mesh,
        scratch_types=[
            pltpu.VMEM((CNT_PER_W, CHUNK), jnp.int32),
            pltpu.VMEM((CHUNK, D), jnp.float32),
            pltpu.VMEM_SHARED((NACC, D), jnp.float32),
            pltpu.SemaphoreType.DMA,
        ],
        name="sc_degcount",
    )


_SQRT1_2 = 0.7071067811865476


def _gelu(z):
    return 0.5 * z * (1.0 + lax.erf(z * _SQRT1_2))


def _embed_body(x_ref, iso_ref, isoemb_ref, win_ref, bin_ref, out_ref):
    x = x_ref[...]
    t = isoemb_ref[...] @ win_ref[D:D + EMB, :]
    oh = (iso_ref[...] == lax.broadcasted_iota(jnp.int32, (BN, ISO), 1))
    h0 = x @ win_ref[:D, :] + oh.astype(jnp.float32) @ t + bin_ref[...]
    out_ref[0] = h0[:, :D]
    out_ref[1] = h0[:, D:]


def _layer_body(hp_ref, s_ref, cnt_ref, wl_ref, bl_ref, wr_ref, g_ref, be_ref,
                out_ref):
    hp = jnp.concatenate([hp_ref[0], hp_ref[1]], axis=-1)
    sv = jnp.concatenate([s_ref[0], s_ref[1]], axis=-1)
    cnt = cnt_ref[0][:, 0:1] + cnt_ref[1][:, 0:1]
    agg = sv / jnp.maximum(cnt, 1.0)
    z = agg @ wl_ref[...] + bl_ref[...] + hp @ wr_ref[...]
    m = jnp.mean(z, axis=-1, keepdims=True)
    v = jnp.mean((z - m) ** 2, axis=-1, keepdims=True)
    z = (z - m) * lax.rsqrt(v + 1e-5) * g_ref[...] + be_ref[...]
    res = hp + _gelu(z)
    out_ref[0] = res[:, :D]
    out_ref[1] = res[:, D:]


def _head_body(hp_ref, wh1_ref, bh1_ref, gh_ref, beh_ref, wh2_ref, bh2_ref,
               out_ref):
    hp = jnp.concatenate([hp_ref[0], hp_ref[1]], axis=-1)
    z = hp @ wh1_ref[...] + bh1_ref[...]
    m = jnp.mean(z, axis=-1, keepdims=True)
    v = jnp.mean((z - m) ** 2, axis=-1, keepdims=True)
    z = (z - m) * lax.rsqrt(v + 1e-5) * gh_ref[...] + beh_ref[...]
    out_ref[...] = _gelu(z) @ wh2_ref[...] + bh2_ref[...]


def _stacked_spec():
    return pl.BlockSpec((2, BN, D), lambda i: (0, i, 0))


def _full_spec(shape):
    nd = len(shape)
    return pl.BlockSpec(shape, lambda i: (0,) * nd)


_embed_call = pl.pallas_call(
    _embed_body,
    grid=(GRID,),
    in_specs=[
        pl.BlockSpec((BN, D), lambda i: (i, 0)),
        pl.BlockSpec((BN, 1), lambda i: (i, 0)),
        _full_spec((ISO, EMB)),
        _full_spec((D + EMB, H)),
        _full_spec((1, H)),
    ],
    out_specs=_stacked_spec(),
    out_shape=jax.ShapeDtypeStruct((2, N, D), jnp.float32),
)

_layer_call = pl.pallas_call(
    _layer_body,
    grid=(GRID,),
    in_specs=[
        _stacked_spec(),
        _stacked_spec(),
        pl.BlockSpec((2, BN, 8), lambda i: (0, i, 0)),
        _full_spec((H, H)),
        _full_spec((1, H)),
        _full_spec((H, H)),
        _full_spec((1, H)),
        _full_spec((1, H)),
    ],
    out_specs=_stacked_spec(),
    out_shape=jax.ShapeDtypeStruct((2, N, D), jnp.float32),
)

_head_call = pl.pallas_call(
    _head_body,
    grid=(GRID,),
    in_specs=[
        _stacked_spec(),
        _full_spec((H, D)),
        _full_spec((1, D)),
        _full_spec((1, D)),
        _full_spec((1, D)),
        _full_spec((D, C)),
        _full_spec((1, C)),
    ],
    out_specs=pl.BlockSpec((BN, C), lambda i: (i, 0)),
    out_shape=jax.ShapeDtypeStruct((N, C), jnp.float32),
)

_seg = _make_seg_kernel()
_cnt = _make_cnt_kernel()


def kernel(x, edge_index, iso_idx, iso_embed, W_in, b_in,
           Wl1, bl1, Wr1, g1, be1, Wl2, bl2, Wr2, g2, be2,
           Wl3, bl3, Wr3, g3, be3, Wl4, bl4, Wr4, g4, be4,
           Wh1, bh1, gh, beh, Wh2, bh2):
    src = edge_index[0].astype(jnp.int32)
    dst = edge_index[1].astype(jnp.int32)
    pad = EP - E
    src_p = jnp.concatenate([src, jnp.zeros((pad,), jnp.int32)])
    srcr = jnp.concatenate([src_p, src_p + N]).reshape(2 * NCHUNKS, CHUNK)
    dstr = jnp.concatenate([dst, jnp.full((pad,), N, jnp.int32)]).reshape(
        NCHUNKS, CHUNK)
    zrows = jnp.zeros((ZROWS, D), jnp.float32)
    ones128 = jnp.ones((CHUNK, D), jnp.float32)

    iso2d = iso_idx.astype(jnp.int32).reshape(N, 1)
    b_in2 = b_in.reshape(1, H)

    h = _embed_call(x, iso2d, iso_embed, W_in, b_in2)
    cnt_full = _cnt(dstr, zrows, ones128)
    cnt = cnt_full[:, :, :8]

    layer_params = [
        (Wl1, bl1, Wr1, g1, be1),
        (Wl2, bl2, Wr2, g2, be2),
        (Wl3, bl3, Wr3, g3, be3),
        (Wl4, bl4, Wr4, g4, be4),
    ]
    for (Wl, bl, Wr, g, be) in layer_params:
        h2 = h.reshape(2 * N, D)
        s = _seg(h2, srcr, dstr, zrows)
        h = _layer_call(h, s, cnt, Wl, bl.reshape(1, H), Wr,
                        g.reshape(1, H), be.reshape(1, H))

    return _head_call(h, Wh1, bh1.reshape(1, D), gh.reshape(1, D),
                      beh.reshape(1, D), Wh2, bh2.reshape(1, C))
